# baseline (speedup 1.0000x reference)
import functools

import jax
import jax.numpy as jnp
from jax.experimental import pallas as pl
from jax.experimental.pallas import tpu as pltpu

DIM = 1024
NUM_EXPERTS = 8
TOP_K = 2
LN_EPS = 1e-5


def _moe_body(x_ref, wr_ref, br_ref, lng_ref, lnb_ref, we_ref, be_ref,
              noise_ref, out_ref, w_sc, wcol_sc):
    e = pl.program_id(1)
    E = NUM_EXPERTS

    @pl.when(e == 0)
    def _routing():
        R = 512

        def body(i, _):
            r0 = pl.multiple_of(i * R, R)
            logits = jax.lax.dot(
                x_ref[pl.ds(r0, R), :], wr_ref[...],
                preferred_element_type=jnp.float32) + br_ref[...]
            mu = jnp.mean(logits, axis=-1, keepdims=True)
            dev = logits - mu
            var = jnp.mean(dev * dev, axis=-1, keepdims=True)
            ln = dev / jnp.sqrt(var + LN_EPS) * lng_ref[...] + lnb_ref[...]
            z = ln - jnp.max(ln, axis=-1, keepdims=True)
            p = jnp.exp(z)
            rw = (p / jnp.sum(p, axis=-1, keepdims=True)
                  + noise_ref[pl.ds(r0, R), :])
            lanes = jax.lax.broadcasted_iota(jnp.int32, rw.shape, 1)
            m1 = jnp.max(rw, axis=-1, keepdims=True)
            i1 = jnp.min(jnp.where(rw == m1, lanes, E), axis=-1,
                         keepdims=True)
            rw2 = jnp.where(lanes == i1, -jnp.inf, rw)
            m2 = jnp.max(rw2, axis=-1, keepdims=True)
            i2 = jnp.min(jnp.where(rw2 == m2, lanes, E), axis=-1,
                         keepdims=True)
            e2 = jnp.exp(m2 - m1)
            s = 1.0 + e2
            w1 = 1.0 / s
            w2 = e2 / s
            w_full = (jnp.where(lanes == i1, w1, 0.0)
                      + jnp.where(lanes == i2, w2, 0.0))
            w_sc[pl.ds(r0, R), :] = w_full
            for ee in range(E):
                wcol_sc[ee, pl.ds(r0, R), :] = w_full[:, ee:ee + 1]
            return 0

        jax.lax.fori_loop(0, x_ref.shape[0] // R, body, 0)

    w_col = wcol_sc[e, :, :]

    y = jax.lax.dot(x_ref[...], we_ref[0],
                    preferred_element_type=jnp.float32)
    contrib = y * w_col

    @pl.when(e == 0)
    def _init():
        out_ref[...] = contrib + jax.lax.dot(
            w_sc[...], be_ref[...],
            preferred_element_type=jnp.float32)

    @pl.when(e != 0)
    def _acc():
        out_ref[...] += contrib


@functools.partial(jax.jit, static_argnames=("t_blk",))
def _moe(x2d, W_r, b_r, ln_g, ln_b, W_e, b_e, noise, t_blk=2048):
    N, D = x2d.shape
    E = W_e.shape[0]
    grid = (N // t_blk, E)
    return pl.pallas_call(
        _moe_body,
        grid=grid,
        in_specs=[
            pl.BlockSpec((t_blk, D), lambda t, e: (t, 0)),
            pl.BlockSpec((D, E), lambda t, e: (0, 0)),
            pl.BlockSpec((1, E), lambda t, e: (0, 0)),
            pl.BlockSpec((1, E), lambda t, e: (0, 0)),
            pl.BlockSpec((1, E), lambda t, e: (0, 0)),
            pl.BlockSpec((1, D, D), lambda t, e: (e, 0, 0)),
            pl.BlockSpec((E, D), lambda t, e: (0, 0)),
            pl.BlockSpec((t_blk, E), lambda t, e: (t, 0)),
        ],
        out_specs=pl.BlockSpec((t_blk, D), lambda t, e: (t, 0)),
        out_shape=jax.ShapeDtypeStruct((N, D), jnp.float32),
        scratch_shapes=[pltpu.VMEM((t_blk, E), jnp.float32),
                        pltpu.VMEM((E, t_blk, 1), jnp.float32)],
        compiler_params=pltpu.CompilerParams(
            dimension_semantics=("arbitrary", "arbitrary"),
            vmem_limit_bytes=100 * 1024 * 1024,
        ),
    )(x2d, W_r, b_r, ln_g, ln_b, W_e, b_e, noise)


def kernel(x, W_r, b_r, ln_g, ln_b, W_e, b_e):
    B, S, D = x.shape
    E = W_e.shape[0]
    noise = jax.random.normal(
        jax.random.key(1), (B, S, E), dtype=jnp.float32) * (1.0 / E)
    out = _moe(
        x.reshape(B * S, D), W_r,
        b_r.reshape(1, E), ln_g.reshape(1, E), ln_b.reshape(1, E),
        W_e, b_e, noise.reshape(B * S, E))
    return out.reshape(B, S, D)


# speedup vs baseline: 1.0106x; 1.0106x over previous
import functools

import jax
import jax.numpy as jnp
from jax.experimental import pallas as pl
from jax.experimental.pallas import tpu as pltpu

DIM = 1024
NUM_EXPERTS = 8
TOP_K = 2
LN_EPS = 1e-5


def _moe_body(x_ref, wr_ref, br_ref, lng_ref, lnb_ref, we_ref, be_ref,
              noise_ref, out_ref, w_sc, wcol_sc):
    e = pl.program_id(1)
    E = NUM_EXPERTS

    @pl.when(e == 0)
    def _routing():
        logits = jax.lax.dot(
            x_ref[...], wr_ref[...],
            preferred_element_type=jnp.float32) + br_ref[...]
        mu = jnp.mean(logits, axis=-1, keepdims=True)
        dev = logits - mu
        var = jnp.mean(dev * dev, axis=-1, keepdims=True)
        ln = dev / jnp.sqrt(var + LN_EPS) * lng_ref[...] + lnb_ref[...]
        z = ln - jnp.max(ln, axis=-1, keepdims=True)
        p = jnp.exp(z)
        rw = p / jnp.sum(p, axis=-1, keepdims=True) + noise_ref[...]
        lanes = jax.lax.broadcasted_iota(jnp.int32, rw.shape, 1)
        m1 = jnp.max(rw, axis=-1, keepdims=True)
        i1 = jnp.min(jnp.where(rw == m1, lanes, E), axis=-1, keepdims=True)
        rw2 = jnp.where(lanes == i1, -jnp.inf, rw)
        m2 = jnp.max(rw2, axis=-1, keepdims=True)
        i2 = jnp.min(jnp.where(rw2 == m2, lanes, E), axis=-1, keepdims=True)
        e2 = jnp.exp(m2 - m1)
        s = 1.0 + e2
        w1 = 1.0 / s
        w2 = e2 / s
        w_full = (jnp.where(lanes == i1, w1, 0.0)
                  + jnp.where(lanes == i2, w2, 0.0))
        w_sc[...] = w_full
        for ee in range(E):
            wcol_sc[ee, :, :] = w_full[:, ee:ee + 1]

    w_col = wcol_sc[e, :, :]

    y = jax.lax.dot(x_ref[...], we_ref[0],
                    preferred_element_type=jnp.float32)
    contrib = y * w_col

    @pl.when(e == 0)
    def _init():
        out_ref[...] = contrib + jax.lax.dot(
            w_sc[...], be_ref[...],
            preferred_element_type=jnp.float32)

    @pl.when(e != 0)
    def _acc():
        out_ref[...] += contrib


@functools.partial(jax.jit, static_argnames=("t_blk",))
def _moe(x2d, W_r, b_r, ln_g, ln_b, W_e, b_e, noise, t_blk=1024):
    N, D = x2d.shape
    E = W_e.shape[0]
    grid = (N // t_blk, E)
    return pl.pallas_call(
        _moe_body,
        grid=grid,
        in_specs=[
            pl.BlockSpec((t_blk, D), lambda t, e: (t, 0)),
            pl.BlockSpec((D, E), lambda t, e: (0, 0)),
            pl.BlockSpec((1, E), lambda t, e: (0, 0)),
            pl.BlockSpec((1, E), lambda t, e: (0, 0)),
            pl.BlockSpec((1, E), lambda t, e: (0, 0)),
            pl.BlockSpec((1, D, D), lambda t, e: (e, 0, 0)),
            pl.BlockSpec((E, D), lambda t, e: (0, 0)),
            pl.BlockSpec((t_blk, E), lambda t, e: (t, 0)),
        ],
        out_specs=pl.BlockSpec((t_blk, D), lambda t, e: (t, 0)),
        out_shape=jax.ShapeDtypeStruct((N, D), jnp.float32),
        scratch_shapes=[pltpu.VMEM((t_blk, E), jnp.float32),
                        pltpu.VMEM((E, t_blk, 1), jnp.float32)],
        compiler_params=pltpu.CompilerParams(
            dimension_semantics=("arbitrary", "arbitrary"),
            vmem_limit_bytes=100 * 1024 * 1024,
        ),
    )(x2d, W_r, b_r, ln_g, ln_b, W_e, b_e, noise)


def kernel(x, W_r, b_r, ln_g, ln_b, W_e, b_e):
    B, S, D = x.shape
    E = W_e.shape[0]
    noise = jax.random.normal(
        jax.random.key(1), (B, S, E), dtype=jnp.float32) * (1.0 / E)
    out = _moe(
        x.reshape(B * S, D), W_r,
        b_r.reshape(1, E), ln_g.reshape(1, E), ln_b.reshape(1, E),
        W_e, b_e, noise.reshape(B * S, E))
    return out.reshape(B, S, D)
